# baseline (speedup 1.0000x reference)
import jax
import jax.numpy as jnp
from jax import lax
from jax.experimental import pallas as pl
from jax.experimental.pallas import tpu as pltpu
from jax.experimental.pallas import tpu_sc as plsc

_B = 4096
_F = 26
_V = 100000
_D = 16
_EPS = 1e-3

_NC = 2
_NS = 16
_NW = _NC * _NS
_BPW = _B // _NW
_L = 16

_RPF = 12504


_TPF = _V // 128
_NU = _F * _TPF
_KB = 8


def _repack_body(a_hbm, c_hbm, in_v, out_v, tin_v, tout_v, sem_i, sem_o):
    wid = lax.axis_index("s") * _NC + lax.axis_index("c")
    per_w = (_NU + _NW - 1) // _NW
    u0 = wid * per_w
    u1 = jnp.minimum(u0 + per_w, _NU)

    lanes = lax.broadcasted_iota(jnp.int32, (_L,), 0)

    def fire_in(u, slot):
        f = u // _TPF
        t = u % _TPF
        v0 = pl.multiple_of(t * 128, 128)
        return pltpu.async_copy(
            a_hbm.at[f, :, pl.ds(v0, 128)], in_v.at[slot], sem_i)

    def fire_out(u, slot):
        f = u // _TPF
        t = u % _TPF
        row0 = pl.multiple_of(f * _RPF + t * _L, 8)
        return pltpu.async_copy(
            out_v.at[slot], c_hbm.at[pl.ds(row0, _L)], sem_o)

    def permute(src, dst, ncols):
        for r in range(ncols // 8):
            for k in range(8):
                col = jnp.full((_L,), 8 * r + k, jnp.int32)
                e = plsc.load_gather(src, [lanes, col])
                dst[r, pl.ds(k * _D, _D)] = e

    def batch_body(b, carry):
        ub = u0 + b * _KB
        for slot in range(_KB):
            u = ub + slot

            @pl.when(u < u1)
            def _():
                fire_in(u, slot)

        for slot in range(_KB):
            u = ub + slot

            @pl.when(u < u1)
            def _():
                pltpu.make_async_copy(
                    a_hbm.at[0, :, pl.ds(0, 128)], in_v.at[slot],
                    sem_i).wait()
                permute(in_v.at[slot], out_v.at[slot], 128)
                fire_out(u, slot)

        for slot in range(_KB):
            u = ub + slot

            @pl.when(u < u1)
            def _():
                pltpu.make_async_copy(
                    out_v.at[slot], c_hbm.at[pl.ds(0, _L)], sem_o).wait()
        return carry

    nb = (per_w + _KB - 1) // _KB
    lax.fori_loop(0, nb, batch_body, 0)

    @pl.when(wid < _F)
    def _():
        v0 = _TPF * 128
        row0 = pl.multiple_of(wid * _RPF + _TPF * _L, 8)
        pltpu.async_copy(
            a_hbm.at[wid, :, pl.ds(v0, _V - v0)], tin_v, sem_i).wait()
        permute(tin_v, tout_v, _V - v0)
        pltpu.async_copy(
            tout_v, c_hbm.at[pl.ds(row0, (_V - v0) // 8)], sem_o).wait()


@jax.jit
def _sc_repack(a3):
    mesh = plsc.VectorSubcoreMesh(core_axis_name="c", subcore_axis_name="s")
    return pl.kernel(
        _repack_body,
        out_type=jax.ShapeDtypeStruct((_F * _RPF, 128), jnp.float32),
        mesh=mesh,
        scratch_types=[
            pltpu.VMEM((_KB, _D, 128), jnp.float32),
            pltpu.VMEM((_KB, _L, 128), jnp.float32),
            pltpu.VMEM((_D, 32), jnp.float32),
            pltpu.VMEM((4, 128), jnp.float32),
            pltpu.SemaphoreType.DMA,
            pltpu.SemaphoreType.DMA,
        ],
        compiler_params=pltpu.CompilerParams(use_tc_tiling_on_sc=True,
                                             needs_layout_passes=False),
    )(a3)


def _sc_body(tbl_hbm, idx_hbm, bi_hbm, idx_v, vt_v, rows_v, sT_v, sqT_v,
             bi_v, sem):
    wid = lax.axis_index("s") * _NC + lax.axis_index("c")
    base_b = wid * _BPW

    pltpu.sync_copy(idx_hbm.at[:, pl.ds(base_b, _BPW)], idx_v)

    def vt_body(j, carry):
        f = j // (_BPW // _L)
        g = j % (_BPW // _L)
        o = pl.multiple_of(g * _L, _L)
        v = idx_v[f, pl.ds(o, _L)]
        vt_v[f, pl.ds(o, _L)] = (
            lax.shift_right_logical(v, 3) + f * _RPF)
        return carry

    lax.fori_loop(0, _F * (_BPW // _L), vt_body, 0)

    def z_body(j, carry):
        r = j // (_BPW // _L)
        g = j % (_BPW // _L)
        o = pl.multiple_of(g * _L, _L)
        zero = jnp.zeros((_L,), jnp.float32)
        sT_v[r, pl.ds(o, _L)] = zero
        sqT_v[r, pl.ds(o, _L)] = zero
        return carry

    lax.fori_loop(0, _D * (_BPW // _L), z_body, 0)

    def fire(f, slot):
        return pltpu.async_copy(
            tbl_hbm.at[vt_v.at[f]],
            rows_v.at[slot],
            sem,
        )

    lanes = lax.broadcasted_iota(jnp.int32, (_L,), 0)

    def extract(f, slot):
        def g_body(g, carry):
            o = pl.multiple_of(g * _L, _L)
            vj = idx_v[f, pl.ds(o, _L)]
            sub = (vj & 7) * _D
            jvec = g * _L + lanes
            for d in range(_D):
                ed = plsc.load_gather(rows_v.at[slot], [jvec, sub + d])
                sT_v[d, pl.ds(o, _L)] = sT_v[d, pl.ds(o, _L)] + ed
                sqT_v[d, pl.ds(o, _L)] = sqT_v[d, pl.ds(o, _L)] + ed * ed
            return carry

        lax.fori_loop(0, _BPW // _L, g_body, 0)

    descs = [fire(0, 0)]
    for f in range(_F):
        if f + 1 < _F:
            descs.append(fire(f + 1, (f + 1) % 2))
        descs[f].wait()
        extract(f, f % 2)

    rows16 = lax.broadcasted_iota(jnp.int32, (_L,), 0)

    def bi_body(b, carry):
        col = jnp.full((_L,), b, jnp.int32)
        s = plsc.load_gather(sT_v, [rows16, col])
        sq = plsc.load_gather(sqT_v, [rows16, col])
        bi_v[b, :] = 0.5 * (s * s - sq)
        return carry

    lax.fori_loop(0, _BPW, bi_body, 0)

    pltpu.sync_copy(bi_v, bi_hbm.at[pl.ds(base_b, _BPW)])


@jax.jit
def _sc_gather_pool(tbl, idxT):
    mesh = plsc.VectorSubcoreMesh(core_axis_name="c", subcore_axis_name="s")
    return pl.kernel(
        _sc_body,
        out_type=jax.ShapeDtypeStruct((_B, _D), jnp.float32),
        mesh=mesh,
        scratch_types=[
            pltpu.VMEM((_F, _BPW), jnp.int32),
            pltpu.VMEM((_F, _BPW), jnp.int32),
            pltpu.VMEM((2, _BPW, 128), jnp.float32),
            pltpu.VMEM((_D, _BPW), jnp.float32),
            pltpu.VMEM((_D, _BPW), jnp.float32),
            pltpu.VMEM((_BPW, _D), jnp.float32),
            pltpu.SemaphoreType.DMA,
        ],
        compiler_params=pltpu.CompilerParams(use_tc_tiling_on_sc=True,
                                             needs_layout_passes=False),
    )(tbl, idxT)


def _tc_body(bi_ref, gamma_ref, beta_ref, W1_ref, b1_ref, W2_ref, b2_ref,
             W3_ref, b3_ref, Wo_ref, bo_ref, out_ref):
    bi = bi_ref[...]
    mean = jnp.mean(bi, axis=0, keepdims=True)
    var = jnp.mean((bi - mean) ** 2, axis=0, keepdims=True)
    x = (bi - mean) * lax.rsqrt(var + _EPS) * gamma_ref[...] + beta_ref[...]
    x = jnp.maximum(jnp.dot(x, W1_ref[...],
                            preferred_element_type=jnp.float32)
                    + b1_ref[...], 0.0)
    x = jnp.maximum(jnp.dot(x, W2_ref[...],
                            preferred_element_type=jnp.float32)
                    + b2_ref[...], 0.0)
    x = jnp.maximum(jnp.dot(x, W3_ref[...],
                            preferred_element_type=jnp.float32)
                    + b3_ref[...], 0.0)
    z = jnp.dot(x, Wo_ref[...], preferred_element_type=jnp.float32) \
        + bo_ref[...]
    out_ref[...] = 1.0 / (1.0 + jnp.exp(-z))


@jax.jit
def _tc_bn_mlp(bi, gamma, beta, W1, b1, W2, b2, W3, b3, Wo, bo):
    return pl.pallas_call(
        _tc_body,
        out_shape=jax.ShapeDtypeStruct((_B, 1), jnp.float32),
    )(bi, gamma, beta, W1, b1, W2, b2, W3, b3, Wo, bo)


def kernel(inputs, tables, gamma, beta, W1, b1, W2, b2, W3, b3, Wo, bo):
    a3 = tables.transpose(0, 2, 1)
    tbl = _sc_repack(a3)
    idxT = inputs.T
    bi = _sc_gather_pool(tbl, idxT)
    return _tc_bn_mlp(
        bi, gamma.reshape(1, _D), beta.reshape(1, _D),
        W1, b1.reshape(1, -1), W2, b2.reshape(1, -1),
        W3, b3.reshape(1, -1), Wo, bo.reshape(1, 1),
    )


# speedup vs baseline: 4.4164x; 4.4164x over previous
import jax
import jax.numpy as jnp
from jax import lax
from jax.experimental import pallas as pl
from jax.experimental.pallas import tpu as pltpu
from jax.experimental.pallas import tpu_sc as plsc

_B = 4096
_F = 26
_V = 100000
_D = 16
_EPS = 1e-3

_NC = 2
_NS = 16
_NW = _NC * _NS
_BPW = _B // _NW
_L = 16

_W = 2048
_NCH = -(-_V // _W)
_VP = _NCH * _W
_NG = 4


def _tr_body(a_ref, o_ref):
    o_ref[...] = a_ref[...].T


@jax.jit
def _tc_transpose(a2):
    return pl.pallas_call(
        _tr_body,
        grid=(_NG, _NCH),
        in_specs=[pl.BlockSpec((8 * _D, _W), lambda g, c: (g, c))],
        out_specs=pl.BlockSpec((_W, 8 * _D),
                               lambda g, c: (g * _NCH + c, 0)),
        out_shape=jax.ShapeDtypeStruct((_NG * _VP, 8 * _D), jnp.float32),
    )(a2)


def _sc_body(tbl_hbm, idx_hbm, bi_hbm, idx_v, vt_v, rows_v, s_v, sq_v,
             bi_v, sem):
    wid = lax.axis_index("s") * _NC + lax.axis_index("c")
    base_b = wid * _BPW

    pltpu.sync_copy(idx_hbm.at[:, pl.ds(base_b, _BPW)], idx_v)

    def vt_body(j, carry):
        f = j // (_BPW // _L)
        g = j % (_BPW // _L)
        o = pl.multiple_of(g * _L, _L)
        vt_v[f, pl.ds(o, _L)] = idx_v[f, pl.ds(o, _L)] + (f // 8) * _VP
        return carry

    lax.fori_loop(0, _F * (_BPW // _L), vt_body, 0)

    def z_body(b, carry):
        zero = jnp.zeros((_L,), jnp.float32)
        s_v[b, :] = zero
        sq_v[b, :] = zero
        return carry

    lax.fori_loop(0, _BPW, z_body, 0)

    def fire(f, slot):
        return pltpu.async_copy(
            tbl_hbm.at[vt_v.at[f]],
            rows_v.at[slot],
            sem,
        )

    def accumulate(f, slot):
        sub = (f % 8) * _D

        def j_body(j, carry):
            e = rows_v[slot, j, pl.ds(sub, _D)]
            s_v[j, :] = s_v[j, :] + e
            sq_v[j, :] = sq_v[j, :] + e * e
            return carry

        lax.fori_loop(0, _BPW, j_body, 0)

    descs = [fire(0, 0)]
    for f in range(_F):
        if f + 1 < _F:
            descs.append(fire(f + 1, (f + 1) % 2))
        descs[f].wait()
        accumulate(f, f % 2)

    def bi_body(b, carry):
        s = s_v[b, :]
        bi_v[b, :] = 0.5 * (s * s - sq_v[b, :])
        return carry

    lax.fori_loop(0, _BPW, bi_body, 0)

    pltpu.sync_copy(bi_v, bi_hbm.at[pl.ds(base_b, _BPW)])


@jax.jit
def _sc_gather_pool(tbl, idxT):
    mesh = plsc.VectorSubcoreMesh(core_axis_name="c", subcore_axis_name="s")
    return pl.kernel(
        _sc_body,
        out_type=jax.ShapeDtypeStruct((_B, _D), jnp.float32),
        mesh=mesh,
        scratch_types=[
            pltpu.VMEM((_F, _BPW), jnp.int32),
            pltpu.VMEM((_F, _BPW), jnp.int32),
            pltpu.VMEM((2, _BPW, 128), jnp.float32),
            pltpu.VMEM((_BPW, _D), jnp.float32),
            pltpu.VMEM((_BPW, _D), jnp.float32),
            pltpu.VMEM((_BPW, _D), jnp.float32),
            pltpu.SemaphoreType.DMA,
        ],
        compiler_params=pltpu.CompilerParams(use_tc_tiling_on_sc=True,
                                             needs_layout_passes=False),
    )(tbl, idxT)


def _tc_body(bi_ref, gamma_ref, beta_ref, W1_ref, b1_ref, W2_ref, b2_ref,
             W3_ref, b3_ref, Wo_ref, bo_ref, out_ref):
    bi = bi_ref[...]
    mean = jnp.mean(bi, axis=0, keepdims=True)
    var = jnp.mean((bi - mean) ** 2, axis=0, keepdims=True)
    x = (bi - mean) * lax.rsqrt(var + _EPS) * gamma_ref[...] + beta_ref[...]
    x = jnp.maximum(jnp.dot(x, W1_ref[...],
                            preferred_element_type=jnp.float32)
                    + b1_ref[...], 0.0)
    x = jnp.maximum(jnp.dot(x, W2_ref[...],
                            preferred_element_type=jnp.float32)
                    + b2_ref[...], 0.0)
    x = jnp.maximum(jnp.dot(x, W3_ref[...],
                            preferred_element_type=jnp.float32)
                    + b3_ref[...], 0.0)
    z = jnp.dot(x, Wo_ref[...], preferred_element_type=jnp.float32) \
        + bo_ref[...]
    out_ref[...] = 1.0 / (1.0 + jnp.exp(-z))


@jax.jit
def _tc_bn_mlp(bi, gamma, beta, W1, b1, W2, b2, W3, b3, Wo, bo):
    return pl.pallas_call(
        _tc_body,
        out_shape=jax.ShapeDtypeStruct((_B, 1), jnp.float32),
    )(bi, gamma, beta, W1, b1, W2, b2, W3, b3, Wo, bo)


def kernel(inputs, tables, gamma, beta, W1, b1, W2, b2, W3, b3, Wo, bo):
    a2 = tables.transpose(0, 2, 1).reshape(_F * _D, _V)
    tbl = _tc_transpose(a2)
    idxT = inputs.T
    bi = _sc_gather_pool(tbl, idxT)
    return _tc_bn_mlp(
        bi, gamma.reshape(1, _D), beta.reshape(1, _D),
        W1, b1.reshape(1, -1), W2, b2.reshape(1, -1),
        W3, b3.reshape(1, -1), Wo, bo.reshape(1, 1),
    )


# speedup vs baseline: 6.3054x; 1.4277x over previous
import functools

import jax
import jax.numpy as jnp
from jax import lax
from jax.experimental import pallas as pl
from jax.experimental.pallas import tpu as pltpu
from jax.experimental.pallas import tpu_sc as plsc

_B = 4096
_F = 26
_V = 100000
_D = 16
_EPS = 1e-3

_NC = 2
_NS = 16
_NW = _NC * _NS
_BPW = _B // _NW
_L = 16

_W = 12800
_NCH = -(-_V // _W)
_VP = _NCH * _W
_NG = 4


def _tr_body(a_ref, o_ref):
    o_ref[...] = a_ref[...].T


def _tc_transpose_group(a2, g):
    return pl.pallas_call(
        _tr_body,
        grid=(_NCH,),
        in_specs=[pl.BlockSpec((8 * _D, _W), lambda c, g=g: (g, c))],
        out_specs=pl.BlockSpec((_W, 8 * _D), lambda c: (c, 0)),
        out_shape=jax.ShapeDtypeStruct((_VP, 8 * _D), jnp.float32),
    )(a2)


def _sc_body(nf, tbl_hbm, idx_hbm, s_hbm, sq_hbm, idx_v, rows_v, s_v,
             sq_v, sem):
    wid = lax.axis_index("s") * _NC + lax.axis_index("c")
    base_b = wid * _BPW

    pltpu.sync_copy(idx_hbm.at[:, pl.ds(base_b, _BPW)], idx_v)

    def z_body(b, carry):
        zero = jnp.zeros((_L,), jnp.float32)
        s_v[b, :] = zero
        sq_v[b, :] = zero
        return carry

    lax.fori_loop(0, _BPW, z_body, 0)

    def fire(f, slot):
        return pltpu.async_copy(
            tbl_hbm.at[idx_v.at[f]],
            rows_v.at[slot],
            sem,
        )

    def accumulate(f, slot):
        sub = f * _D

        def j_body(j, carry):
            e = rows_v[slot, j, pl.ds(sub, _D)]
            s_v[j, :] = s_v[j, :] + e
            sq_v[j, :] = sq_v[j, :] + e * e
            return carry

        lax.fori_loop(0, _BPW, j_body, 0)

    descs = [fire(0, 0)]
    for f in range(nf):
        if f + 1 < nf:
            descs.append(fire(f + 1, (f + 1) % 2))
        descs[f].wait()
        accumulate(f, f % 2)

    pltpu.sync_copy(s_v, s_hbm.at[pl.ds(base_b, _BPW)])
    pltpu.sync_copy(sq_v, sq_hbm.at[pl.ds(base_b, _BPW)])


def _sc_gather_group(tbl, idxTg, nf):
    mesh = plsc.VectorSubcoreMesh(core_axis_name="c", subcore_axis_name="s")
    return pl.kernel(
        functools.partial(_sc_body, nf),
        out_type=(jax.ShapeDtypeStruct((_B, _D), jnp.float32),
                  jax.ShapeDtypeStruct((_B, _D), jnp.float32)),
        mesh=mesh,
        scratch_types=[
            pltpu.VMEM((nf, _BPW), jnp.int32),
            pltpu.VMEM((2, _BPW, 128), jnp.float32),
            pltpu.VMEM((_BPW, _D), jnp.float32),
            pltpu.VMEM((_BPW, _D), jnp.float32),
            pltpu.SemaphoreType.DMA,
        ],
        compiler_params=pltpu.CompilerParams(use_tc_tiling_on_sc=True,
                                             needs_layout_passes=False),
    )(tbl, idxTg)


def _tc_body(s0, s1, s2, s3, q0, q1, q2, q3, gamma_ref, beta_ref,
             W1_ref, b1_ref, W2_ref, b2_ref, W3_ref, b3_ref, Wo_ref,
             bo_ref, out_ref):
    s = s0[...] + s1[...] + s2[...] + s3[...]
    sq = q0[...] + q1[...] + q2[...] + q3[...]
    bi = 0.5 * (s * s - sq)
    mean = jnp.mean(bi, axis=0, keepdims=True)
    var = jnp.mean((bi - mean) ** 2, axis=0, keepdims=True)
    x = (bi - mean) * lax.rsqrt(var + _EPS) * gamma_ref[...] + beta_ref[...]
    x = jnp.maximum(jnp.dot(x, W1_ref[...],
                            preferred_element_type=jnp.float32)
                    + b1_ref[...], 0.0)
    x = jnp.maximum(jnp.dot(x, W2_ref[...],
                            preferred_element_type=jnp.float32)
                    + b2_ref[...], 0.0)
    x = jnp.maximum(jnp.dot(x, W3_ref[...],
                            preferred_element_type=jnp.float32)
                    + b3_ref[...], 0.0)
    z = jnp.dot(x, Wo_ref[...], preferred_element_type=jnp.float32) \
        + bo_ref[...]
    out_ref[...] = 1.0 / (1.0 + jnp.exp(-z))


@jax.jit
def _nfm(a2, idxT, gamma, beta, W1, b1, W2, b2, W3, b3, Wo, bo):
    parts = []
    for g in range(_NG):
        nf = min(8, _F - 8 * g)
        tbl_g = _tc_transpose_group(a2, g)
        parts.append(_sc_gather_group(tbl_g, idxT[8 * g:8 * g + nf], nf))
    ss = [p[0] for p in parts]
    qq = [p[1] for p in parts]
    return pl.pallas_call(
        _tc_body,
        out_shape=jax.ShapeDtypeStruct((_B, 1), jnp.float32),
    )(*ss, *qq, gamma, beta, W1, b1, W2, b2, W3, b3, Wo, bo)


def kernel(inputs, tables, gamma, beta, W1, b1, W2, b2, W3, b3, Wo, bo):
    a2 = tables.transpose(0, 2, 1).reshape(_F * _D, _V)
    idxT = inputs.T
    return _nfm(
        a2, idxT, gamma.reshape(1, _D), beta.reshape(1, _D),
        W1, b1.reshape(1, -1), W2, b2.reshape(1, -1),
        W3, b3.reshape(1, -1), Wo, bo.reshape(1, 1),
    )


# speedup vs baseline: 6.5740x; 1.0426x over previous
import jax
import jax.numpy as jnp
from jax import lax
from jax.experimental import pallas as pl
from jax.experimental.pallas import tpu as pltpu
from jax.experimental.pallas import tpu_sc as plsc

_B = 4096
_F = 26
_V = 100000
_D = 16
_EPS = 1e-3

_NC = 2
_NS = 16
_NW = _NC * _NS
_BPW = _B // _NW
_L = 16

_W = 12800
_NCH = -(-_V // _W)
_VP = _NCH * _W
_NG = 4


def _tr_body(a_ref, o_ref):
    o_ref[...] = a_ref[...].T


@jax.jit
def _tc_transpose(a2):
    return pl.pallas_call(
        _tr_body,
        grid=(_NG, _NCH),
        in_specs=[pl.BlockSpec((8 * _D, _W), lambda g, c: (g, c))],
        out_specs=pl.BlockSpec((_W, 8 * _D),
                               lambda g, c: (g * _NCH + c, 0)),
        out_shape=jax.ShapeDtypeStruct((_NG * _VP, 8 * _D), jnp.float32),
    )(a2)


def _sc_body(tbl_hbm, idx_hbm, bi_hbm, idx_v, vt_v, rows_v, s_v, sq_v,
             bi_v, sem):
    wid = lax.axis_index("s") * _NC + lax.axis_index("c")
    base_b = wid * _BPW

    pltpu.sync_copy(idx_hbm.at[:, pl.ds(base_b, _BPW)], idx_v)

    def vt_body(j, carry):
        f = j // (_BPW // _L)
        g = j % (_BPW // _L)
        o = pl.multiple_of(g * _L, _L)
        vt_v[f, pl.ds(o, _L)] = idx_v[f, pl.ds(o, _L)] + (f // 8) * _VP
        return carry

    lax.fori_loop(0, _F * (_BPW // _L), vt_body, 0)

    def z_body(b, carry):
        zero = jnp.zeros((_L,), jnp.float32)
        s_v[b, :] = zero
        sq_v[b, :] = zero
        return carry

    lax.fori_loop(0, _BPW, z_body, 0)

    def fire(f, slot):
        return pltpu.async_copy(
            tbl_hbm.at[vt_v.at[f]],
            rows_v.at[slot],
            sem,
        )

    def accumulate(f, slot):
        sub = (f % 8) * _D

        def j_body(j, carry):
            e = rows_v[slot, j, pl.ds(sub, _D)]
            s_v[j, :] = s_v[j, :] + e
            sq_v[j, :] = sq_v[j, :] + e * e
            return carry

        lax.fori_loop(0, _BPW, j_body, 0)

    descs = [fire(0, 0)]
    for f in range(_F):
        if f + 1 < _F:
            descs.append(fire(f + 1, (f + 1) % 2))
        descs[f].wait()
        accumulate(f, f % 2)

    def bi_body(b, carry):
        s = s_v[b, :]
        bi_v[b, :] = 0.5 * (s * s - sq_v[b, :])
        return carry

    lax.fori_loop(0, _BPW, bi_body, 0)

    pltpu.sync_copy(bi_v, bi_hbm.at[pl.ds(base_b, _BPW)])


@jax.jit
def _sc_gather_pool(tbl, idxT):
    mesh = plsc.VectorSubcoreMesh(core_axis_name="c", subcore_axis_name="s")
    return pl.kernel(
        _sc_body,
        out_type=jax.ShapeDtypeStruct((_B, _D), jnp.float32),
        mesh=mesh,
        scratch_types=[
            pltpu.VMEM((_F, _BPW), jnp.int32),
            pltpu.VMEM((_F, _BPW), jnp.int32),
            pltpu.VMEM((2, _BPW, 128), jnp.float32),
            pltpu.VMEM((_BPW, _D), jnp.float32),
            pltpu.VMEM((_BPW, _D), jnp.float32),
            pltpu.VMEM((_BPW, _D), jnp.float32),
            pltpu.SemaphoreType.DMA,
        ],
        compiler_params=pltpu.CompilerParams(use_tc_tiling_on_sc=True,
                                             needs_layout_passes=False),
    )(tbl, idxT)


def _tc_body(bi_ref, gamma_ref, beta_ref, W1_ref, b1_ref, W2_ref, b2_ref,
             W3_ref, b3_ref, Wo_ref, bo_ref, out_ref):
    bi = bi_ref[...]
    mean = jnp.mean(bi, axis=0, keepdims=True)
    var = jnp.mean((bi - mean) ** 2, axis=0, keepdims=True)
    x = (bi - mean) * lax.rsqrt(var + _EPS) * gamma_ref[...] + beta_ref[...]
    x = jnp.maximum(jnp.dot(x, W1_ref[...],
                            preferred_element_type=jnp.float32)
                    + b1_ref[...], 0.0)
    x = jnp.maximum(jnp.dot(x, W2_ref[...],
                            preferred_element_type=jnp.float32)
                    + b2_ref[...], 0.0)
    x = jnp.maximum(jnp.dot(x, W3_ref[...],
                            preferred_element_type=jnp.float32)
                    + b3_ref[...], 0.0)
    z = jnp.dot(x, Wo_ref[...], preferred_element_type=jnp.float32) \
        + bo_ref[...]
    out_ref[...] = 1.0 / (1.0 + jnp.exp(-z))


@jax.jit
def _tc_bn_mlp(bi, gamma, beta, W1, b1, W2, b2, W3, b3, Wo, bo):
    return pl.pallas_call(
        _tc_body,
        out_shape=jax.ShapeDtypeStruct((_B, 1), jnp.float32),
    )(bi, gamma, beta, W1, b1, W2, b2, W3, b3, Wo, bo)


def kernel(inputs, tables, gamma, beta, W1, b1, W2, b2, W3, b3, Wo, bo):
    a2 = tables.transpose(0, 2, 1).reshape(_F * _D, _V)
    tbl = _tc_transpose(a2)
    idxT = inputs.T
    bi = _sc_gather_pool(tbl, idxT)
    return _tc_bn_mlp(
        bi, gamma.reshape(1, _D), beta.reshape(1, _D),
        W1, b1.reshape(1, -1), W2, b2.reshape(1, -1),
        W3, b3.reshape(1, -1), Wo, bo.reshape(1, 1),
    )
